# Initial kernel scaffold; baseline (speedup 1.0000x reference)
import jax
import jax.numpy as jnp
from jax.experimental import pallas as pl


def kernel(x, edge_index, W0, b0, W1, b1, W2, b2, ln0_g, ln0_b, ln1_g, ln1_b):
    raise NotImplementedError("write your pallas kernel here")


# speedup vs baseline: 22.9066x; 22.9066x over previous
import jax
import jax.numpy as jnp
from jax import lax
from jax.experimental import pallas as pl
from jax.experimental.pallas import tpu as pltpu
from jax.experimental.pallas import tpu_sc as plsc

N = 10000
E = 320000
D = 128
NC = 2
NS = 16
NW = NC * NS
CHUNK = 96
GK = 6
NCHUNK = 108
NGRP = NCHUNK // GK
EPT = NCHUNK * CHUNK
EPAD = EPT * NW
NPAD = 10240
SLAB = NPAD // NS

_MESH = plsc.VectorSubcoreMesh(
    core_axis_name="c", subcore_axis_name="s", num_cores=NC, num_subcores=NS
)


def _agg_body(t_hbm, srcg_hbm, dstg_hbm, zD_hbm, out_hbm,
              acc, srcg, dstg, rows, rsem0, rsem1, rsem2, isem):
    c = lax.axis_index("c")
    s = lax.axis_index("s")
    r0 = s * SLAB
    pltpu.sync_copy(zD_hbm, acc.at[pl.ds(r0, SLAB)])
    pltpu.sync_copy(srcg_hbm.at[c, s, 0], srcg.at[0])
    pltpu.sync_copy(dstg_hbm.at[c, s, 0], dstg.at[0])
    pltpu.async_copy(srcg_hbm.at[c, s, 1], srcg.at[1], isem)
    pltpu.async_copy(dstg_hbm.at[c, s, 1], dstg.at[1], isem)
    plsc.subcore_barrier()

    rsems = (rsem0, rsem1, rsem2)
    for b in range(3):
        pltpu.async_copy(t_hbm.at[srcg.at[0, b]], rows.at[b], rsems[b])

    def group(g, carry):
        gp = lax.rem(g, 2)
        gp1 = 1 - gp

        @pl.when(g < NGRP - 1)
        def _():
            pltpu.make_async_copy(srcg_hbm.at[c, s, g + 1], srcg.at[gp1], isem).wait()
            pltpu.make_async_copy(dstg_hbm.at[c, s, g + 1], dstg.at[gp1], isem).wait()

        for jj in range(GK):
            b = jj % 3
            pltpu.make_async_copy(
                t_hbm.at[srcg.at[gp, jj]], rows.at[b], rsems[b]).wait()
            pltpu.sync_copy(rows.at[b], acc.at[dstg.at[gp, jj]], add=True)
            if jj <= GK - 4:
                pltpu.async_copy(t_hbm.at[srcg.at[gp, jj + 3]], rows.at[b], rsems[b])
            else:
                @pl.when(g < NGRP - 1)
                def _():
                    pltpu.async_copy(
                        t_hbm.at[srcg.at[gp1, jj + 3 - GK]], rows.at[b], rsems[b])

        @pl.when(g < NGRP - 2)
        def _():
            pltpu.async_copy(srcg_hbm.at[c, s, g + 2], srcg.at[gp], isem)
            pltpu.async_copy(dstg_hbm.at[c, s, g + 2], dstg.at[gp], isem)

        return carry

    lax.fori_loop(0, NGRP, group, 0)
    plsc.subcore_barrier()
    pltpu.sync_copy(acc.at[pl.ds(r0, SLAB)], out_hbm.at[c, pl.ds(r0, SLAB)])


_agg_call = pl.kernel(
    _agg_body,
    out_type=jax.ShapeDtypeStruct((NC, NPAD, D), jnp.float32),
    mesh=_MESH,
    scratch_types=[
        pltpu.VMEM_SHARED((NPAD, D), jnp.float32),
        pltpu.VMEM((2, GK, CHUNK), jnp.int32),
        pltpu.VMEM((2, GK, CHUNK), jnp.int32),
        pltpu.VMEM((3, CHUNK, D), jnp.float32),
        pltpu.SemaphoreType.DMA,
        pltpu.SemaphoreType.DMA,
        pltpu.SemaphoreType.DMA,
        pltpu.SemaphoreType.DMA,
    ],
)


def _deg_body(dst_hbm, ones_hbm, zD_hbm, out_hbm, acc, dstv, onesv):
    c = lax.axis_index("c")
    s = lax.axis_index("s")
    r0 = s * SLAB
    pltpu.sync_copy(zD_hbm, acc.at[pl.ds(r0, SLAB)])
    pltpu.sync_copy(dst_hbm.at[c, s], dstv)
    pltpu.sync_copy(ones_hbm, onesv)
    plsc.subcore_barrier()

    def step(j, carry):
        pltpu.sync_copy(onesv, acc.at[dstv.at[j]], add=True)
        return carry

    lax.fori_loop(0, NCHUNK, step, 0)
    plsc.subcore_barrier()
    pltpu.sync_copy(acc.at[pl.ds(r0, SLAB)], out_hbm.at[c, pl.ds(r0, SLAB)])


_deg_call = pl.kernel(
    _deg_body,
    out_type=jax.ShapeDtypeStruct((NC, NPAD, D), jnp.float32),
    mesh=_MESH,
    scratch_types=[
        pltpu.VMEM_SHARED((NPAD, D), jnp.float32),
        pltpu.VMEM((NCHUNK, CHUNK), jnp.int32),
        pltpu.VMEM((CHUNK, D), jnp.float32),
    ],
)


_RB = 1024


def _row_mask(i):
    rid = lax.broadcasted_iota(jnp.int32, (_RB, 1), 0) + i * _RB
    return rid < N


def _tc0_body(degp_ref, x_ref, dinv_ref, t0_ref):
    deg = degp_ref[0, :, 0:1] + degp_ref[1, :, 0:1] + 1.0
    di = lax.rsqrt(deg)
    dinv_ref[...] = di
    t0_ref[...] = jnp.where(_row_mask(pl.program_id(0)), x_ref[...] * di, 0.0)


def _tc0(degp, x):
    return pl.pallas_call(
        _tc0_body,
        grid=(NPAD // _RB,),
        in_specs=[
            pl.BlockSpec((NC, _RB, D), lambda i: (0, i, 0)),
            pl.BlockSpec((_RB, D), lambda i: (i, 0)),
        ],
        out_specs=[
            pl.BlockSpec((_RB, 1), lambda i: (i, 0)),
            pl.BlockSpec((_RB, D), lambda i: (i, 0)),
        ],
        out_shape=[
            jax.ShapeDtypeStruct((N, 1), jnp.float32),
            jax.ShapeDtypeStruct((NPAD, D), jnp.float32),
        ],
    )(degp, x)


def _layer_body(s_ref, t_ref, h_ref, dinv_ref, w_ref, b_ref, g_ref, lb_ref,
                h_out, t_out):
    di = dinv_ref[...]
    agg = (s_ref[0] + s_ref[1] + t_ref[...]) * di
    conv = jnp.dot(agg, w_ref[...], preferred_element_type=jnp.float32)
    a = jnp.maximum(conv + b_ref[...], 0.0)
    mu = jnp.mean(a, axis=1, keepdims=True)
    var = jnp.mean((a - mu) ** 2, axis=1, keepdims=True)
    h = (a - mu) * lax.rsqrt(var + 1e-5) * g_ref[...] + lb_ref[...] + h_ref[...]
    h_out[...] = h
    t_out[...] = jnp.where(_row_mask(pl.program_id(0)), h * di, 0.0)


def _layer(s2, t, h, dinv, w, b, g, lb):
    return pl.pallas_call(
        _layer_body,
        grid=(NPAD // _RB,),
        in_specs=[
            pl.BlockSpec((NC, _RB, D), lambda i: (0, i, 0)),
            pl.BlockSpec((_RB, D), lambda i: (i, 0)),
            pl.BlockSpec((_RB, D), lambda i: (i, 0)),
            pl.BlockSpec((_RB, 1), lambda i: (i, 0)),
            pl.BlockSpec((D, D), lambda i: (0, 0)),
            pl.BlockSpec((1, D), lambda i: (0, 0)),
            pl.BlockSpec((1, D), lambda i: (0, 0)),
            pl.BlockSpec((1, D), lambda i: (0, 0)),
        ],
        out_specs=[
            pl.BlockSpec((_RB, D), lambda i: (i, 0)),
            pl.BlockSpec((_RB, D), lambda i: (i, 0)),
        ],
        out_shape=[
            jax.ShapeDtypeStruct((N, D), jnp.float32),
            jax.ShapeDtypeStruct((NPAD, D), jnp.float32),
        ],
    )(s2, t, h, dinv, w, b, g, lb)


def _final_body(s_ref, t_ref, dinv_ref, w_ref, b_ref, out_ref):
    agg = (s_ref[0] + s_ref[1] + t_ref[...]) * dinv_ref[...]
    out_ref[...] = jnp.dot(agg, w_ref[...], preferred_element_type=jnp.float32) + b_ref[...]


def _final(s2, t, dinv, w, b):
    return pl.pallas_call(
        _final_body,
        grid=(NPAD // _RB,),
        in_specs=[
            pl.BlockSpec((NC, _RB, D), lambda i: (0, i, 0)),
            pl.BlockSpec((_RB, D), lambda i: (i, 0)),
            pl.BlockSpec((_RB, 1), lambda i: (i, 0)),
            pl.BlockSpec((D, 1), lambda i: (0, 0)),
            pl.BlockSpec((1, 1), lambda i: (0, 0)),
        ],
        out_specs=pl.BlockSpec((_RB, 1), lambda i: (i, 0)),
        out_shape=jax.ShapeDtypeStruct((N, 1), jnp.float32),
    )(s2, t, dinv, w, b)


@jax.jit
def kernel(x, edge_index, W0, b0, W1, b1, W2, b2, ln0_g, ln0_b, ln1_g, ln1_b):
    pad = N + (jnp.arange(EPAD - E, dtype=jnp.int32) % (NPAD - N))
    srcp = jnp.concatenate([edge_index[0], pad])
    dstp = jnp.concatenate([edge_index[1], pad])
    src = srcp.reshape(NC, NS, NGRP, GK, CHUNK)
    dst = dstp.reshape(NC, NS, NGRP, GK, CHUNK)
    dst4 = dstp.reshape(NC, NS, NCHUNK, CHUNK)
    zD = jnp.zeros((SLAB, D), jnp.float32)
    onesrows = jnp.ones((CHUNK, D), jnp.float32)

    degp = _deg_call(dst4, onesrows, zD)
    dinv, t0 = _tc0(degp, x)
    s0 = _agg_call(t0, src, dst, zD)
    h1, t1 = _layer(s0, t0, x, dinv, W0, b0.reshape(1, D),
                    ln0_g.reshape(1, D), ln0_b.reshape(1, D))
    s1 = _agg_call(t1, src, dst, zD)
    h2, t2 = _layer(s1, t1, h1, dinv, W1, b1.reshape(1, D),
                    ln1_g.reshape(1, D), ln1_b.reshape(1, D))
    s2 = _agg_call(t2, src, dst, zD)
    return _final(s2, t2, dinv, W2, b2.reshape(1, 1))


# speedup vs baseline: 25.4933x; 1.1129x over previous
import jax
import jax.numpy as jnp
from jax import lax
from jax.experimental import pallas as pl
from jax.experimental.pallas import tpu as pltpu
from jax.experimental.pallas import tpu_sc as plsc

N = 10000
E = 320000
D = 128
NC = 2
NS = 16
NW = NC * NS
CHUNK = 96
GK = 6
NCHUNK = 108
NGRP = NCHUNK // GK
EPT = NCHUNK * CHUNK
EPAD = EPT * NW
NPAD = 10240
SLAB = NPAD // NS

_MESH = plsc.VectorSubcoreMesh(
    core_axis_name="c", subcore_axis_name="s", num_cores=NC, num_subcores=NS
)


def _agg_body(t_hbm, srcg_hbm, dstg_hbm, zD_hbm, out_hbm,
              acc, srcg, dstg, rows, rsem0, rsem1, rsem2, isem):
    c = lax.axis_index("c")
    s = lax.axis_index("s")
    r0 = s * SLAB
    pltpu.sync_copy(zD_hbm, acc.at[pl.ds(r0, SLAB)])
    pltpu.sync_copy(srcg_hbm.at[c, s, 0], srcg.at[0])
    pltpu.sync_copy(dstg_hbm.at[c, s, 0], dstg.at[0])
    pltpu.async_copy(srcg_hbm.at[c, s, 1], srcg.at[1], isem)
    pltpu.async_copy(dstg_hbm.at[c, s, 1], dstg.at[1], isem)
    plsc.subcore_barrier()

    rsems = (rsem0, rsem1, rsem2)
    for b in range(3):
        pltpu.async_copy(t_hbm.at[srcg.at[0, b]], rows.at[b], rsems[b])

    def group(g, carry):
        gp = lax.rem(g, 2)
        gp1 = 1 - gp

        @pl.when(g < NGRP - 1)
        def _():
            pltpu.make_async_copy(srcg_hbm.at[c, s, g + 1], srcg.at[gp1], isem).wait()
            pltpu.make_async_copy(dstg_hbm.at[c, s, g + 1], dstg.at[gp1], isem).wait()

        for jj in range(GK):
            b = jj % 3
            pltpu.make_async_copy(
                t_hbm.at[srcg.at[gp, jj]], rows.at[b], rsems[b]).wait()
            pltpu.sync_copy(rows.at[b], acc.at[dstg.at[gp, jj]], add=True)
            if jj <= GK - 4:
                pltpu.async_copy(t_hbm.at[srcg.at[gp, jj + 3]], rows.at[b], rsems[b])
            else:
                @pl.when(g < NGRP - 1)
                def _():
                    pltpu.async_copy(
                        t_hbm.at[srcg.at[gp1, jj + 3 - GK]], rows.at[b], rsems[b])

        @pl.when(g < NGRP - 2)
        def _():
            pltpu.async_copy(srcg_hbm.at[c, s, g + 2], srcg.at[gp], isem)
            pltpu.async_copy(dstg_hbm.at[c, s, g + 2], dstg.at[gp], isem)

        return carry

    lax.fori_loop(0, NGRP, group, 0)
    plsc.subcore_barrier()
    pltpu.sync_copy(acc.at[pl.ds(r0, SLAB)], out_hbm.at[c, pl.ds(r0, SLAB)])


_agg_call = pl.kernel(
    _agg_body,
    out_type=jax.ShapeDtypeStruct((NC, NPAD, D), jnp.float32),
    mesh=_MESH,
    scratch_types=[
        pltpu.VMEM_SHARED((NPAD, D), jnp.float32),
        pltpu.VMEM((2, GK, CHUNK), jnp.int32),
        pltpu.VMEM((2, GK, CHUNK), jnp.int32),
        pltpu.VMEM((3, CHUNK, D), jnp.float32),
        pltpu.SemaphoreType.DMA,
        pltpu.SemaphoreType.DMA,
        pltpu.SemaphoreType.DMA,
        pltpu.SemaphoreType.DMA,
    ],
)


def _deg_body(dst_hbm, ones_hbm, z16_hbm, out_hbm, acc, dstv, onesv):
    c = lax.axis_index("c")
    s = lax.axis_index("s")
    r0 = s * SLAB
    pltpu.sync_copy(z16_hbm, acc.at[pl.ds(r0, SLAB)])
    pltpu.sync_copy(dst_hbm.at[c, s], dstv)
    pltpu.sync_copy(ones_hbm, onesv)
    plsc.subcore_barrier()

    def step(j, carry):
        pltpu.sync_copy(onesv, acc.at[dstv.at[j]], add=True)
        return carry

    lax.fori_loop(0, NCHUNK, step, 0)
    plsc.subcore_barrier()
    pltpu.sync_copy(acc.at[pl.ds(r0, SLAB)], out_hbm.at[c, pl.ds(r0, SLAB)])


_deg_call = pl.kernel(
    _deg_body,
    out_type=jax.ShapeDtypeStruct((NC, NPAD, 16), jnp.float32),
    mesh=_MESH,
    compiler_params=pltpu.CompilerParams(use_tc_tiling_on_sc=False),
    scratch_types=[
        pltpu.VMEM_SHARED((NPAD, 16), jnp.float32),
        pltpu.VMEM((NCHUNK, CHUNK), jnp.int32),
        pltpu.VMEM((CHUNK, 16), jnp.float32),
    ],
)


_RB = 1024


def _row_mask(i):
    rid = lax.broadcasted_iota(jnp.int32, (_RB, 1), 0) + i * _RB
    return rid < N


def _tc0_body(degp_ref, x_ref, dinv_ref, t0_ref):
    deg = degp_ref[0, :, 0:1] + degp_ref[1, :, 0:1] + 1.0
    di = lax.rsqrt(deg)
    dinv_ref[...] = di
    t0_ref[...] = jnp.where(_row_mask(pl.program_id(0)), x_ref[...] * di, 0.0)


def _tc0(degp, x):
    return pl.pallas_call(
        _tc0_body,
        grid=(NPAD // _RB,),
        in_specs=[
            pl.BlockSpec((NC, _RB, 16), lambda i: (0, i, 0)),
            pl.BlockSpec((_RB, D), lambda i: (i, 0)),
        ],
        out_specs=[
            pl.BlockSpec((_RB, 1), lambda i: (i, 0)),
            pl.BlockSpec((_RB, D), lambda i: (i, 0)),
        ],
        out_shape=[
            jax.ShapeDtypeStruct((N, 1), jnp.float32),
            jax.ShapeDtypeStruct((NPAD, D), jnp.float32),
        ],
    )(degp, x)


def _layer_body(s_ref, t_ref, h_ref, dinv_ref, w_ref, b_ref, g_ref, lb_ref,
                h_out, t_out):
    di = dinv_ref[...]
    agg = (s_ref[0] + s_ref[1] + t_ref[...]) * di
    conv = jnp.dot(agg, w_ref[...], preferred_element_type=jnp.float32)
    a = jnp.maximum(conv + b_ref[...], 0.0)
    mu = jnp.mean(a, axis=1, keepdims=True)
    var = jnp.mean((a - mu) ** 2, axis=1, keepdims=True)
    h = (a - mu) * lax.rsqrt(var + 1e-5) * g_ref[...] + lb_ref[...] + h_ref[...]
    h_out[...] = h
    t_out[...] = jnp.where(_row_mask(pl.program_id(0)), h * di, 0.0)


def _layer(s2, t, h, dinv, w, b, g, lb):
    return pl.pallas_call(
        _layer_body,
        grid=(NPAD // _RB,),
        in_specs=[
            pl.BlockSpec((NC, _RB, D), lambda i: (0, i, 0)),
            pl.BlockSpec((_RB, D), lambda i: (i, 0)),
            pl.BlockSpec((_RB, D), lambda i: (i, 0)),
            pl.BlockSpec((_RB, 1), lambda i: (i, 0)),
            pl.BlockSpec((D, D), lambda i: (0, 0)),
            pl.BlockSpec((1, D), lambda i: (0, 0)),
            pl.BlockSpec((1, D), lambda i: (0, 0)),
            pl.BlockSpec((1, D), lambda i: (0, 0)),
        ],
        out_specs=[
            pl.BlockSpec((_RB, D), lambda i: (i, 0)),
            pl.BlockSpec((_RB, D), lambda i: (i, 0)),
        ],
        out_shape=[
            jax.ShapeDtypeStruct((N, D), jnp.float32),
            jax.ShapeDtypeStruct((NPAD, D), jnp.float32),
        ],
    )(s2, t, h, dinv, w, b, g, lb)


def _final_body(s_ref, t_ref, dinv_ref, w_ref, b_ref, out_ref):
    agg = (s_ref[0] + s_ref[1] + t_ref[...]) * dinv_ref[...]
    out_ref[...] = jnp.dot(agg, w_ref[...], preferred_element_type=jnp.float32) + b_ref[...]


def _final(s2, t, dinv, w, b):
    return pl.pallas_call(
        _final_body,
        grid=(NPAD // _RB,),
        in_specs=[
            pl.BlockSpec((NC, _RB, D), lambda i: (0, i, 0)),
            pl.BlockSpec((_RB, D), lambda i: (i, 0)),
            pl.BlockSpec((_RB, 1), lambda i: (i, 0)),
            pl.BlockSpec((D, 1), lambda i: (0, 0)),
            pl.BlockSpec((1, 1), lambda i: (0, 0)),
        ],
        out_specs=pl.BlockSpec((_RB, 1), lambda i: (i, 0)),
        out_shape=jax.ShapeDtypeStruct((N, 1), jnp.float32),
    )(s2, t, dinv, w, b)


@jax.jit
def kernel(x, edge_index, W0, b0, W1, b1, W2, b2, ln0_g, ln0_b, ln1_g, ln1_b):
    pad = N + (jnp.arange(EPAD - E, dtype=jnp.int32) % (NPAD - N))
    srcp = jnp.concatenate([edge_index[0], pad])
    dstp = jnp.concatenate([edge_index[1], pad])
    src = srcp.reshape(NC, NS, NGRP, GK, CHUNK)
    dst = dstp.reshape(NC, NS, NGRP, GK, CHUNK)
    dst4 = dstp.reshape(NC, NS, NCHUNK, CHUNK)
    zD = jnp.zeros((SLAB, D), jnp.float32)
    ones16 = jnp.ones((CHUNK, 16), jnp.float32)
    z16 = jnp.zeros((SLAB, 16), jnp.float32)

    degp = _deg_call(dst4, ones16, z16)
    dinv, t0 = _tc0(degp, x)
    s0 = _agg_call(t0, src, dst, zD)
    h1, t1 = _layer(s0, t0, x, dinv, W0, b0.reshape(1, D),
                    ln0_g.reshape(1, D), ln0_b.reshape(1, D))
    s1 = _agg_call(t1, src, dst, zD)
    h2, t2 = _layer(s1, t1, h1, dinv, W1, b1.reshape(1, D),
                    ln1_g.reshape(1, D), ln1_b.reshape(1, D))
    s2 = _agg_call(t2, src, dst, zD)
    return _final(s2, t2, dinv, W2, b2.reshape(1, 1))
